# baseline (speedup 1.0000x reference)
import jax
import jax.numpy as jnp
from jax import lax
from jax.experimental import pallas as pl
from jax.experimental.pallas import tpu as pltpu
from jax.experimental.pallas import tpu_sc as plsc

_NUM_CORES = 2
_NUM_SUBCORES = 16
_NW = _NUM_CORES * _NUM_SUBCORES

_D = 128
_BATCH = 4
_SEQ = 4096
_N = _BATCH * _SEQ
_PER_W = _N // _NW
_W_PER_B = _NW // _BATCH
_CHUNK = 128
_NCHUNK = _PER_W // _CHUNK


def _embed_body(tok_tab, pos_tab, tid, pid, out,
                tidx_v, pidx_v, rows,
                sem_i0, sem_i1, sem_g0, sem_g1, sem_g2, sem_g3, sem_s):
    c = lax.axis_index("c")
    s = lax.axis_index("s")
    wid = s * _NUM_CORES + c
    row = wid // _W_PER_B
    col = (wid % _W_PER_B) * _PER_W
    sem_g = (sem_g0, sem_g1, sem_g2, sem_g3)

    ci_t = pltpu.async_copy(tid.at[row, pl.ds(col, _PER_W)], tidx_v, sem_i0)
    ci_p = pltpu.async_copy(pid.at[row, pl.ds(col, _PER_W)], pidx_v, sem_i1)

    ci_t.wait()
    toks = [pltpu.async_copy(tok_tab.at[tidx_v.at[pl.ds(j * _CHUNK, _CHUNK)]],
                             rows.at[pl.ds(j * _CHUNK, _CHUNK)], sem_g[j])
            for j in range(_NCHUNK)]
    ci_p.wait()
    adds = []
    for j in range(_NCHUNK):
        toks[j].wait()
        adds.append(pltpu.async_copy(
            pos_tab.at[pidx_v.at[pl.ds(j * _CHUNK, _CHUNK)]],
            rows.at[pl.ds(j * _CHUNK, _CHUNK)], sem_g[j], add=True))
    for ad in adds:
        ad.wait()
    pltpu.async_copy(rows, out.at[row, pl.ds(col, _PER_W)], sem_s).wait()


def _embed(tok_table, pos_table, tid, pid):
    mesh = plsc.VectorSubcoreMesh(core_axis_name="c", subcore_axis_name="s")
    return pl.kernel(
        _embed_body,
        out_type=jax.ShapeDtypeStruct((_BATCH, _SEQ, _D), jnp.float32),
        mesh=mesh,
        scratch_types=[
            pltpu.VMEM((_PER_W,), jnp.int32),
            pltpu.VMEM((_PER_W,), jnp.int32),
            pltpu.VMEM((_PER_W, _D), jnp.float32),
            pltpu.SemaphoreType.DMA,
            pltpu.SemaphoreType.DMA,
            pltpu.SemaphoreType.DMA,
            pltpu.SemaphoreType.DMA,
            pltpu.SemaphoreType.DMA,
            pltpu.SemaphoreType.DMA,
            pltpu.SemaphoreType.DMA,
        ],
    )(tok_table, pos_table, tid, pid)


def kernel(token_ids, position_ids, tok_table, pos_table):
    return _embed(tok_table, pos_table, token_ids, position_ids)
